# Initial kernel scaffold; baseline (speedup 1.0000x reference)
import jax
import jax.numpy as jnp
from jax.experimental import pallas as pl


def kernel(x, edge_index, edge_weight, W, b):
    raise NotImplementedError("write your pallas kernel here")


# speedup vs baseline: 8.2560x; 8.2560x over previous
import functools

import jax
import jax.numpy as jnp
from jax import lax
from jax.experimental import pallas as pl
from jax.experimental.pallas import tpu as pltpu
from jax.experimental.pallas import tpu_sc as plsc

N = 10000
E = 320000
D = 128
C = 40
CP = 48

NC = 2
NS = 16
NW = NC * NS
EW = E // NW
CHUNK = 80
NCHUNK = EW // CHUNK
NP = 10240
RPT = NP // NS


def _matmul_body(x_ref, wt_ref, o_ref):
    o_ref[...] = jnp.dot(x_ref[...], wt_ref[...],
                         preferred_element_type=jnp.float32)


def _tc_matmul(x, wt):
    bm = 1000
    return pl.pallas_call(
        _matmul_body,
        grid=(N // bm,),
        in_specs=[
            pl.BlockSpec((bm, D), lambda i: (i, 0)),
            pl.BlockSpec((D, CP), lambda i: (0, 0)),
        ],
        out_specs=pl.BlockSpec((bm, CP), lambda i: (i, 0)),
        out_shape=jax.ShapeDtypeStruct((N, CP), jnp.float32),
    )(x, wt)


def _combine_body(p_ref, b_ref, o_ref):
    o_ref[...] = p_ref[0] + p_ref[1] + b_ref[...]


def _tc_combine(p, bvec):
    bm = 1024
    return pl.pallas_call(
        _combine_body,
        grid=(NP // bm,),
        in_specs=[
            pl.BlockSpec((2, bm, CP), lambda i: (0, i, 0)),
            pl.BlockSpec((1, CP), lambda i: (0, 0)),
        ],
        out_specs=pl.BlockSpec((bm, CP), lambda i: (i, 0)),
        out_shape=jax.ShapeDtypeStruct((NP, CP), jnp.float32),
    )(p, bvec)


def _sc_round_body(y_hbm, src_hbm, dst_hbm, w_hbm, out_hbm,
                   src_v, dst_v, w_v, rows_v, acc, sem):
    cid = lax.axis_index("c")
    sid = lax.axis_index("s")
    wid = sid * NC + cid

    pltpu.sync_copy(src_hbm.at[wid], src_v)
    pltpu.sync_copy(dst_hbm.at[wid], dst_v)
    pltpu.sync_copy(w_hbm.at[wid], w_v)

    for c in range(CP // 16):
        for r in range(CHUNK):
            rows_v[r, pl.ds(c * 16, 16)] = jnp.zeros((16,), jnp.float32)
    base = sid * RPT
    for i in range(RPT // CHUNK):
        pltpu.sync_copy(rows_v, acc.at[pl.ds(base + i * CHUNK, CHUNK)])
    plsc.subcore_barrier()

    def step(k, carry):
        pltpu.async_copy(y_hbm.at[src_v.at[k]], rows_v, sem).wait()
        for g in range(CHUNK // 16):
            wv = w_v[k, pl.ds(g * 16, 16)]
            for j in range(16):
                e = g * 16 + j
                wj = wv[j]
                for c in range(CP // 16):
                    sl = pl.ds(c * 16, 16)
                    rows_v[e, sl] = rows_v[e, sl] * wj
        pltpu.sync_copy(rows_v, acc.at[dst_v.at[k]], add=True)
        return carry

    lax.fori_loop(0, NCHUNK, step, 0)
    plsc.subcore_barrier()

    pltpu.sync_copy(acc.at[pl.ds(base, RPT)], out_hbm.at[cid, pl.ds(base, RPT)])


_sc_round = pl.kernel(
    _sc_round_body,
    out_type=jax.ShapeDtypeStruct((NC, NP, CP), jnp.float32),
    mesh=plsc.VectorSubcoreMesh(core_axis_name="c", subcore_axis_name="s",
                                num_cores=NC, num_subcores=NS),
    compiler_params=pltpu.CompilerParams(use_tc_tiling_on_sc=False),
    scratch_types=[
        pltpu.VMEM((NCHUNK, CHUNK), jnp.int32),
        pltpu.VMEM((NCHUNK, CHUNK), jnp.int32),
        pltpu.VMEM((NCHUNK, CHUNK), jnp.float32),
        pltpu.VMEM((CHUNK, CP), jnp.float32),
        pltpu.VMEM_SHARED((NP, CP), jnp.float32),
        pltpu.SemaphoreType.DMA,
    ],
)


def kernel(x, edge_index, edge_weight, W, b):
    src = edge_index[0].reshape(NW, NCHUNK, CHUNK)
    dst = edge_index[1].reshape(NW, NCHUNK, CHUNK)
    w3 = edge_weight.reshape(NW, NCHUNK, CHUNK)
    wt = jnp.zeros((D, CP), jnp.float32).at[:, :C].set(W.T)
    bp = jnp.zeros((1, CP), jnp.float32).at[0, :C].set(b)
    zb = jnp.zeros((1, CP), jnp.float32)

    y = _tc_matmul(x, wt)
    p = _sc_round(y, src, dst, w3)
    r = _tc_combine(p, zb)
    q = _sc_round(r, src, dst, w3)
    o = _tc_combine(q, bp)
    return o[:N, :C]


# speedup vs baseline: 12.4765x; 1.5112x over previous
import functools

import jax
import jax.numpy as jnp
from jax import lax
from jax.experimental import pallas as pl
from jax.experimental.pallas import tpu as pltpu
from jax.experimental.pallas import tpu_sc as plsc

N = 10000
E = 320000
D = 128
C = 40
CP = 48

NC = 2
NS = 16
NW = NC * NS
EW = E // NW
CHUNK = 80
NCHUNK = EW // CHUNK
NP = 10240
RPT = NP // NS
NBUF = 3
UNROLL = 5


def _matmul_body(x_ref, wt_ref, o_ref):
    o_ref[...] = jnp.dot(x_ref[...], wt_ref[...],
                         preferred_element_type=jnp.float32)


def _tc_matmul(x, wt):
    bm = 1000
    return pl.pallas_call(
        _matmul_body,
        grid=(N // bm,),
        in_specs=[
            pl.BlockSpec((bm, D), lambda i: (i, 0)),
            pl.BlockSpec((D, CP), lambda i: (0, 0)),
        ],
        out_specs=pl.BlockSpec((bm, CP), lambda i: (i, 0)),
        out_shape=jax.ShapeDtypeStruct((N, CP), jnp.float32),
    )(x, wt)


def _combine_body(p_ref, b_ref, o_ref):
    o_ref[...] = p_ref[0] + p_ref[1] + b_ref[...]


def _tc_combine(p, bvec):
    bm = 1024
    return pl.pallas_call(
        _combine_body,
        grid=(NP // bm,),
        in_specs=[
            pl.BlockSpec((2, bm, CP), lambda i: (0, i, 0)),
            pl.BlockSpec((1, CP), lambda i: (0, 0)),
        ],
        out_specs=pl.BlockSpec((bm, CP), lambda i: (i, 0)),
        out_shape=jax.ShapeDtypeStruct((NP, CP), jnp.float32),
    )(p, bvec)


def _sc_round_body(y_hbm, src_hbm, dst_hbm, w_hbm, out_hbm,
                   src_v, dst_v, w_v, rows_v, acc, semg, sems):
    cid = lax.axis_index("c")
    sid = lax.axis_index("s")
    wid = sid * NC + cid

    pltpu.sync_copy(src_hbm.at[wid], src_v)
    pltpu.sync_copy(dst_hbm.at[wid], dst_v)
    pltpu.sync_copy(w_hbm.at[wid], w_v)

    for c in range(CP // 16):
        for r in range(CHUNK):
            rows_v[0, r, pl.ds(c * 16, 16)] = jnp.zeros((16,), jnp.float32)
    base = sid * RPT
    for i in range(RPT // CHUNK):
        pltpu.sync_copy(rows_v.at[0], acc.at[pl.ds(base + i * CHUNK, CHUNK)])
    plsc.subcore_barrier()

    def g_start(c, b):
        return pltpu.async_copy(y_hbm.at[src_v.at[c]], rows_v.at[b],
                                semg.at[b])

    def s_start(c, b):
        return pltpu.async_copy(rows_v.at[b], acc.at[dst_v.at[c]],
                                sems.at[b], add=True)

    def scale(c, b):
        for g in range(CHUNK // 16):
            wv = w_v[c, pl.ds(g * 16, 16)]
            for j in range(16):
                e = g * 16 + j
                wj = wv[j]
                for cc in range(CP // 16):
                    sl = pl.ds(cc * 16, 16)
                    rows_v[b, e, sl] = rows_v[b, e, sl] * wj

    def step(i, carry):
        c0 = i * UNROLL
        dg = [g_start(c0 + u, u % NBUF) for u in range(NBUF)]
        ds = [None] * NBUF
        for u in range(UNROLL):
            b = u % NBUF
            if u >= 1 and u - 1 + NBUF < UNROLL:
                pb = (u - 1) % NBUF
                ds[pb].wait()
                dg[pb] = g_start(c0 + u - 1 + NBUF, pb)
            dg[b].wait()
            scale(c0 + u, b)
            ds[b] = s_start(c0 + u, b)
        for b in range(NBUF):
            ds[b].wait()
        return carry

    lax.fori_loop(0, NCHUNK // UNROLL, step, 0)
    plsc.subcore_barrier()

    pltpu.sync_copy(acc.at[pl.ds(base, RPT)], out_hbm.at[cid, pl.ds(base, RPT)])


_sc_round = pl.kernel(
    _sc_round_body,
    out_type=jax.ShapeDtypeStruct((NC, NP, CP), jnp.float32),
    mesh=plsc.VectorSubcoreMesh(core_axis_name="c", subcore_axis_name="s",
                                num_cores=NC, num_subcores=NS),
    compiler_params=pltpu.CompilerParams(use_tc_tiling_on_sc=False),
    scratch_types=[
        pltpu.VMEM((NCHUNK, CHUNK), jnp.int32),
        pltpu.VMEM((NCHUNK, CHUNK), jnp.int32),
        pltpu.VMEM((NCHUNK, CHUNK), jnp.float32),
        pltpu.VMEM((NBUF, CHUNK, CP), jnp.float32),
        pltpu.VMEM_SHARED((NP, CP), jnp.float32),
        pltpu.SemaphoreType.DMA((NBUF,)),
        pltpu.SemaphoreType.DMA((NBUF,)),
    ],
)


def kernel(x, edge_index, edge_weight, W, b):
    src = edge_index[0].reshape(NW, NCHUNK, CHUNK)
    dst = edge_index[1].reshape(NW, NCHUNK, CHUNK)
    w3 = edge_weight.reshape(NW, NCHUNK, CHUNK)
    wt = jnp.zeros((D, CP), jnp.float32).at[:, :C].set(W.T)
    bp = jnp.zeros((1, CP), jnp.float32).at[0, :C].set(b)
    zb = jnp.zeros((1, CP), jnp.float32)

    y = _tc_matmul(x, wt)
    p = _sc_round(y, src, dst, w3)
    r = _tc_combine(p, zb)
    q = _sc_round(r, src, dst, w3)
    o = _tc_combine(q, bp)
    return o[:N, :C]
